# Initial kernel scaffold; baseline (speedup 1.0000x reference)
import jax
import jax.numpy as jnp
from jax.experimental import pallas as pl


def kernel(loss, prev_loss, params, weights, edge_src, edge_dst):
    raise NotImplementedError("write your pallas kernel here")


# speedup vs baseline: 143.5821x; 143.5821x over previous
import functools

import jax
import jax.numpy as jnp
from jax.experimental import pallas as pl
from jax.experimental.pallas import tpu as pltpu

NUM_NODES = 64
NUM_INPUTS = 8
NUM_EDGES = 256
TILE = 4096


def _fused_kernel(oh_dst_ref, oh_src_ref, w_ref, loss_ref, prev_ref, params_ref,
                  out_ref):
    wsrc = oh_src_ref[:] * w_ref[:]
    a = jnp.dot(oh_dst_ref[:], wsrc,
                preferred_element_type=jnp.float32)
    feat = jnp.concatenate([loss_ref[:], prev_ref[:], params_ref[:]], axis=0)
    out_ref[:] = jnp.dot(a, feat, preferred_element_type=jnp.float32)


@jax.jit
def kernel(loss, prev_loss, params, weights, edge_src, edge_dst):
    n = loss.shape[0]
    n_hidden = NUM_NODES - NUM_INPUTS
    oh_dst = (edge_dst[None, :] == (jnp.arange(n_hidden, dtype=jnp.int32)
                                    + NUM_INPUTS)[:, None]).astype(jnp.float32)
    oh_src = (edge_src[:, None]
              == jnp.arange(NUM_INPUTS, dtype=jnp.int32)[None, :]
              ).astype(jnp.float32)
    w2d = weights[:, None]

    grid = (n // TILE,)
    out = pl.pallas_call(
        _fused_kernel,
        grid=grid,
        in_specs=[
            pl.BlockSpec((n_hidden, NUM_EDGES), lambda i: (0, 0)),
            pl.BlockSpec((NUM_EDGES, NUM_INPUTS), lambda i: (0, 0)),
            pl.BlockSpec((NUM_EDGES, 1), lambda i: (0, 0)),
            pl.BlockSpec((1, TILE), lambda i: (0, i)),
            pl.BlockSpec((1, TILE), lambda i: (0, i)),
            pl.BlockSpec((6, TILE), lambda i: (0, i)),
        ],
        out_specs=pl.BlockSpec((n_hidden, TILE), lambda i: (0, i)),
        out_shape=jax.ShapeDtypeStruct((n_hidden, n), jnp.float32),
        compiler_params=pltpu.CompilerParams(
            dimension_semantics=("parallel",)),
    )(oh_dst, oh_src, w2d, loss[None, :], prev_loss[None, :], params)
    return out


# speedup vs baseline: 210.8043x; 1.4682x over previous
import functools

import jax
import jax.numpy as jnp
from jax.experimental import pallas as pl
from jax.experimental.pallas import tpu as pltpu

NUM_NODES = 64
NUM_INPUTS = 8
NUM_EDGES = 256
TILE = 8192


def _fused_kernel(oh_dst_ref, oh_src_ref, w_ref, loss_ref, prev_ref, params_ref,
                  out_ref):
    wsrc = oh_src_ref[:] * w_ref[:]
    a = jnp.dot(oh_dst_ref[:], wsrc,
                preferred_element_type=jnp.float32)
    feat = jnp.concatenate([loss_ref[:], prev_ref[:], params_ref[:]], axis=0)
    out_ref[:] = jnp.dot(a, feat, preferred_element_type=jnp.float32)


@jax.jit
def kernel(loss, prev_loss, params, weights, edge_src, edge_dst):
    n = loss.shape[0]
    n_hidden = NUM_NODES - NUM_INPUTS
    oh_dst = (edge_dst[None, :] == (jnp.arange(n_hidden, dtype=jnp.int32)
                                    + NUM_INPUTS)[:, None]).astype(jnp.float32)
    oh_src = (edge_src[:, None]
              == jnp.arange(NUM_INPUTS, dtype=jnp.int32)[None, :]
              ).astype(jnp.float32)
    w2d = weights[:, None]

    grid = (n // TILE,)
    out = pl.pallas_call(
        _fused_kernel,
        grid=grid,
        in_specs=[
            pl.BlockSpec((n_hidden, NUM_EDGES), lambda i: (0, 0)),
            pl.BlockSpec((NUM_EDGES, NUM_INPUTS), lambda i: (0, 0)),
            pl.BlockSpec((NUM_EDGES, 1), lambda i: (0, 0)),
            pl.BlockSpec((1, TILE), lambda i: (0, i)),
            pl.BlockSpec((1, TILE), lambda i: (0, i)),
            pl.BlockSpec((6, TILE), lambda i: (0, i)),
        ],
        out_specs=pl.BlockSpec((n_hidden, TILE), lambda i: (0, i)),
        out_shape=jax.ShapeDtypeStruct((n_hidden, n), jnp.float32),
        compiler_params=pltpu.CompilerParams(
            dimension_semantics=("parallel",)),
    )(oh_dst, oh_src, w2d, loss[None, :], prev_loss[None, :], params)
    return out


# speedup vs baseline: 274.4599x; 1.3020x over previous
import functools

import jax
import jax.numpy as jnp
from jax.experimental import pallas as pl
from jax.experimental.pallas import tpu as pltpu

NUM_NODES = 64
NUM_INPUTS = 8
NUM_EDGES = 256
TILE = 16384


def _fused_kernel(oh_dst_ref, oh_src_ref, w_ref, loss_ref, prev_ref, params_ref,
                  out_ref):
    wsrc = oh_src_ref[:] * w_ref[:]
    a = jnp.dot(oh_dst_ref[:], wsrc,
                preferred_element_type=jnp.float32)
    feat = jnp.concatenate([loss_ref[:], prev_ref[:], params_ref[:]], axis=0)
    out_ref[:] = jnp.dot(a, feat, preferred_element_type=jnp.float32)


@jax.jit
def kernel(loss, prev_loss, params, weights, edge_src, edge_dst):
    n = loss.shape[0]
    n_hidden = NUM_NODES - NUM_INPUTS
    oh_dst = (edge_dst[None, :] == (jnp.arange(n_hidden, dtype=jnp.int32)
                                    + NUM_INPUTS)[:, None]).astype(jnp.float32)
    oh_src = (edge_src[:, None]
              == jnp.arange(NUM_INPUTS, dtype=jnp.int32)[None, :]
              ).astype(jnp.float32)
    w2d = weights[:, None]

    grid = (n // TILE,)
    out = pl.pallas_call(
        _fused_kernel,
        grid=grid,
        in_specs=[
            pl.BlockSpec((n_hidden, NUM_EDGES), lambda i: (0, 0)),
            pl.BlockSpec((NUM_EDGES, NUM_INPUTS), lambda i: (0, 0)),
            pl.BlockSpec((NUM_EDGES, 1), lambda i: (0, 0)),
            pl.BlockSpec((1, TILE), lambda i: (0, i)),
            pl.BlockSpec((1, TILE), lambda i: (0, i)),
            pl.BlockSpec((6, TILE), lambda i: (0, i)),
        ],
        out_specs=pl.BlockSpec((n_hidden, TILE), lambda i: (0, i)),
        out_shape=jax.ShapeDtypeStruct((n_hidden, n), jnp.float32),
        compiler_params=pltpu.CompilerParams(
            dimension_semantics=("parallel",)),
    )(oh_dst, oh_src, w2d, loss[None, :], prev_loss[None, :], params)
    return out


# speedup vs baseline: 312.3917x; 1.1382x over previous
import functools

import jax
import jax.numpy as jnp
from jax.experimental import pallas as pl
from jax.experimental.pallas import tpu as pltpu

NUM_NODES = 64
NUM_INPUTS = 8
NUM_EDGES = 256
TILE = 32768


def _fused_kernel(oh_dst_ref, oh_src_ref, w_ref, loss_ref, prev_ref, params_ref,
                  out_ref):
    wsrc = oh_src_ref[:] * w_ref[:]
    a = jnp.dot(oh_dst_ref[:], wsrc,
                preferred_element_type=jnp.float32)
    feat = jnp.concatenate([loss_ref[:], prev_ref[:], params_ref[:]], axis=0)
    out_ref[:] = jnp.dot(a, feat, preferred_element_type=jnp.float32)


@jax.jit
def kernel(loss, prev_loss, params, weights, edge_src, edge_dst):
    n = loss.shape[0]
    n_hidden = NUM_NODES - NUM_INPUTS
    oh_dst = (edge_dst[None, :] == (jnp.arange(n_hidden, dtype=jnp.int32)
                                    + NUM_INPUTS)[:, None]).astype(jnp.float32)
    oh_src = (edge_src[:, None]
              == jnp.arange(NUM_INPUTS, dtype=jnp.int32)[None, :]
              ).astype(jnp.float32)
    w2d = weights[:, None]

    grid = (n // TILE,)
    out = pl.pallas_call(
        _fused_kernel,
        grid=grid,
        in_specs=[
            pl.BlockSpec((n_hidden, NUM_EDGES), lambda i: (0, 0)),
            pl.BlockSpec((NUM_EDGES, NUM_INPUTS), lambda i: (0, 0)),
            pl.BlockSpec((NUM_EDGES, 1), lambda i: (0, 0)),
            pl.BlockSpec((1, TILE), lambda i: (0, i)),
            pl.BlockSpec((1, TILE), lambda i: (0, i)),
            pl.BlockSpec((6, TILE), lambda i: (0, i)),
        ],
        out_specs=pl.BlockSpec((n_hidden, TILE), lambda i: (0, i)),
        out_shape=jax.ShapeDtypeStruct((n_hidden, n), jnp.float32),
        compiler_params=pltpu.CompilerParams(
            dimension_semantics=("parallel",)),
    )(oh_dst, oh_src, w2d, loss[None, :], prev_loss[None, :], params)
    return out


# speedup vs baseline: 318.7362x; 1.0203x over previous
import functools

import jax
import jax.numpy as jnp
from jax.experimental import pallas as pl
from jax.experimental.pallas import tpu as pltpu

NUM_NODES = 64
NUM_INPUTS = 8
NUM_EDGES = 256
TILE = 65536


def _fused_kernel(oh_dst_ref, oh_src_ref, w_ref, loss_ref, prev_ref, params_ref,
                  out_ref):
    wsrc = oh_src_ref[:] * w_ref[:]
    a = jnp.dot(oh_dst_ref[:], wsrc,
                preferred_element_type=jnp.float32)
    feat = jnp.concatenate([loss_ref[:], prev_ref[:], params_ref[:]], axis=0)
    out_ref[:] = jnp.dot(a, feat, preferred_element_type=jnp.float32)


@jax.jit
def kernel(loss, prev_loss, params, weights, edge_src, edge_dst):
    n = loss.shape[0]
    n_hidden = NUM_NODES - NUM_INPUTS
    oh_dst = (edge_dst[None, :] == (jnp.arange(n_hidden, dtype=jnp.int32)
                                    + NUM_INPUTS)[:, None]).astype(jnp.float32)
    oh_src = (edge_src[:, None]
              == jnp.arange(NUM_INPUTS, dtype=jnp.int32)[None, :]
              ).astype(jnp.float32)
    w2d = weights[:, None]

    grid = (n // TILE,)
    out = pl.pallas_call(
        _fused_kernel,
        grid=grid,
        in_specs=[
            pl.BlockSpec((n_hidden, NUM_EDGES), lambda i: (0, 0)),
            pl.BlockSpec((NUM_EDGES, NUM_INPUTS), lambda i: (0, 0)),
            pl.BlockSpec((NUM_EDGES, 1), lambda i: (0, 0)),
            pl.BlockSpec((1, TILE), lambda i: (0, i)),
            pl.BlockSpec((1, TILE), lambda i: (0, i)),
            pl.BlockSpec((6, TILE), lambda i: (0, i)),
        ],
        out_specs=pl.BlockSpec((n_hidden, TILE), lambda i: (0, i)),
        out_shape=jax.ShapeDtypeStruct((n_hidden, n), jnp.float32),
        compiler_params=pltpu.CompilerParams(
            dimension_semantics=("parallel",)),
    )(oh_dst, oh_src, w2d, loss[None, :], prev_loss[None, :], params)
    return out


# speedup vs baseline: 319.4873x; 1.0024x over previous
import functools

import jax
import jax.numpy as jnp
from jax.experimental import pallas as pl
from jax.experimental.pallas import tpu as pltpu

NUM_NODES = 64
NUM_INPUTS = 8
NUM_EDGES = 256
TILE = 65536


def _fused_kernel(oh_dst_ref, oh_src_ref, w_ref, loss_ref, prev_ref, params_ref,
                  out_ref, a_ref):
    @pl.when(pl.program_id(0) == 0)
    def _():
        wsrc = oh_src_ref[:] * w_ref[:]
        a_ref[:] = jnp.dot(oh_dst_ref[:], wsrc,
                           preferred_element_type=jnp.float32)

    feat = jnp.concatenate([loss_ref[:], prev_ref[:], params_ref[:]], axis=0)
    out_ref[:] = jnp.dot(a_ref[:], feat, preferred_element_type=jnp.float32)


@jax.jit
def kernel(loss, prev_loss, params, weights, edge_src, edge_dst):
    n = loss.shape[0]
    n_hidden = NUM_NODES - NUM_INPUTS
    oh_dst = (edge_dst[None, :] == (jnp.arange(n_hidden, dtype=jnp.int32)
                                    + NUM_INPUTS)[:, None]).astype(jnp.float32)
    oh_src = (edge_src[:, None]
              == jnp.arange(NUM_INPUTS, dtype=jnp.int32)[None, :]
              ).astype(jnp.float32)
    w2d = weights[:, None]

    grid = (n // TILE,)
    out = pl.pallas_call(
        _fused_kernel,
        grid=grid,
        in_specs=[
            pl.BlockSpec((n_hidden, NUM_EDGES), lambda i: (0, 0)),
            pl.BlockSpec((NUM_EDGES, NUM_INPUTS), lambda i: (0, 0)),
            pl.BlockSpec((NUM_EDGES, 1), lambda i: (0, 0)),
            pl.BlockSpec((1, TILE), lambda i: (0, i)),
            pl.BlockSpec((1, TILE), lambda i: (0, i)),
            pl.BlockSpec((6, TILE), lambda i: (0, i)),
        ],
        out_specs=pl.BlockSpec((n_hidden, TILE), lambda i: (0, i)),
        out_shape=jax.ShapeDtypeStruct((n_hidden, n), jnp.float32),
        scratch_shapes=[pltpu.VMEM((n_hidden, NUM_INPUTS), jnp.float32)],
        compiler_params=pltpu.CompilerParams(
            dimension_semantics=("arbitrary",)),
    )(oh_dst, oh_src, w2d, loss[None, :], prev_loss[None, :], params)
    return out
